# baseline (device time: 19645 ns/iter reference)
import jax
import jax.numpy as jnp
from jax import lax
from jax.experimental import pallas as pl
from jax.experimental.pallas import tpu as pltpu


def kernel(partial, resid, gamma):
    m, d = resid.shape
    gamma2d = gamma.reshape(1, d)

    def body(p_ref, r_ref, g_ref, out_ref, comm_ref, send_sem, recv_sem):
        my_x = lax.axis_index("x")
        my_y = lax.axis_index("y")
        my_z = lax.axis_index("z")
        partner = (my_x, my_y, 1 - my_z)

        barrier_sem = pltpu.get_barrier_semaphore()
        pl.semaphore_signal(
            barrier_sem, inc=1, device_id=partner,
            device_id_type=pl.DeviceIdType.MESH,
        )
        pl.semaphore_wait(barrier_sem, 1)

        rdma = pltpu.make_async_remote_copy(
            src_ref=p_ref,
            dst_ref=comm_ref,
            send_sem=send_sem,
            recv_sem=recv_sem,
            device_id=partner,
            device_id_type=pl.DeviceIdType.MESH,
        )
        rdma.start()
        rdma.wait()

        y = p_ref[0] + comm_ref[0] + r_ref[...]
        rms = jnp.sqrt(jnp.mean(y * y, axis=-1, keepdims=True) + 1e-6)
        out_ref[...] = y / rms * g_ref[...]

    return pl.pallas_call(
        body,
        out_shape=jax.ShapeDtypeStruct((m, d), jnp.float32),
        in_specs=[
            pl.BlockSpec(memory_space=pltpu.VMEM),
            pl.BlockSpec(memory_space=pltpu.VMEM),
            pl.BlockSpec(memory_space=pltpu.VMEM),
        ],
        out_specs=pl.BlockSpec(memory_space=pltpu.VMEM),
        scratch_shapes=[
            pltpu.VMEM((1, m, d), jnp.float32),
            pltpu.SemaphoreType.DMA,
            pltpu.SemaphoreType.DMA,
        ],
        compiler_params=pltpu.CompilerParams(collective_id=0),
    )(partial, resid, gamma2d)


# device time: 17395 ns/iter; 1.1293x vs baseline; 1.1293x over previous
import jax
import jax.numpy as jnp
from jax import lax
from jax.experimental import pallas as pl
from jax.experimental.pallas import tpu as pltpu


def kernel(partial, resid, gamma):
    m, d = resid.shape
    qm = m // 4
    gamma2d = gamma.reshape(1, d)
    partial2d = partial.reshape(m, d)

    def body(p_ref, r_ref, g_ref, out_ref, comm_ref, acc_ref,
             send_sems, recv_sems):
        my_x = lax.axis_index("x")
        my_y = lax.axis_index("y")
        my_z = lax.axis_index("z")
        bx = my_x ^ my_z
        by = my_y ^ my_z
        q = 2 * bx + by
        row0 = q * qm
        partner_z = (my_x, my_y, 1 - my_z)
        partner_x = (1 - my_x, my_y, my_z)
        partner_y = (my_x, 1 - my_y, my_z)

        barrier_sem = pltpu.get_barrier_semaphore()
        for nbr in (partner_z, partner_x, partner_y):
            pl.semaphore_signal(
                barrier_sem, inc=1, device_id=nbr,
                device_id_type=pl.DeviceIdType.MESH,
            )
        pl.semaphore_wait(barrier_sem, 3)

        ph1 = pltpu.make_async_remote_copy(
            src_ref=p_ref.at[pl.ds((3 - q) * qm, qm), :],
            dst_ref=comm_ref,
            send_sem=send_sems.at[0],
            recv_sem=recv_sems.at[0],
            device_id=partner_z,
            device_id_type=pl.DeviceIdType.MESH,
        )
        ph1.start()

        acc_ref[...] = (
            p_ref[pl.ds(row0, qm), :] + r_ref[pl.ds(row0, qm), :]
        )

        ph1.wait_recv()
        y = acc_ref[...] + comm_ref[...]
        rms = jnp.sqrt(jnp.mean(y * y, axis=-1, keepdims=True) + 1e-6)
        out_ref[pl.ds(row0, qm), :] = y / rms * g_ref[...]

        send_q = out_ref.at[pl.ds(row0, qm), :]
        ph2z = pltpu.make_async_remote_copy(
            src_ref=send_q,
            dst_ref=out_ref.at[pl.ds(row0, qm), :],
            send_sem=send_sems.at[1],
            recv_sem=recv_sems.at[1],
            device_id=partner_z,
            device_id_type=pl.DeviceIdType.MESH,
        )
        ph2x = pltpu.make_async_remote_copy(
            src_ref=send_q,
            dst_ref=out_ref.at[pl.ds(row0, qm), :],
            send_sem=send_sems.at[2],
            recv_sem=recv_sems.at[2],
            device_id=partner_x,
            device_id_type=pl.DeviceIdType.MESH,
        )
        ph2y = pltpu.make_async_remote_copy(
            src_ref=send_q,
            dst_ref=out_ref.at[pl.ds(row0, qm), :],
            send_sem=send_sems.at[3],
            recv_sem=recv_sems.at[3],
            device_id=partner_y,
            device_id_type=pl.DeviceIdType.MESH,
        )
        ph2z.start()
        ph2x.start()
        ph2y.start()
        ph1.wait_send()

        rz = pltpu.make_async_remote_copy(
            src_ref=send_q,
            dst_ref=out_ref.at[pl.ds((3 - q) * qm, qm), :],
            send_sem=send_sems.at[1],
            recv_sem=recv_sems.at[1],
            device_id=partner_z,
            device_id_type=pl.DeviceIdType.MESH,
        )
        rx = pltpu.make_async_remote_copy(
            src_ref=send_q,
            dst_ref=out_ref.at[pl.ds((q ^ 2) * qm, qm), :],
            send_sem=send_sems.at[2],
            recv_sem=recv_sems.at[2],
            device_id=partner_x,
            device_id_type=pl.DeviceIdType.MESH,
        )
        ry = pltpu.make_async_remote_copy(
            src_ref=send_q,
            dst_ref=out_ref.at[pl.ds((q ^ 1) * qm, qm), :],
            send_sem=send_sems.at[3],
            recv_sem=recv_sems.at[3],
            device_id=partner_y,
            device_id_type=pl.DeviceIdType.MESH,
        )
        rz.wait_recv()
        rx.wait_recv()
        ry.wait_recv()
        ph2z.wait_send()
        ph2x.wait_send()
        ph2y.wait_send()

    return pl.pallas_call(
        body,
        out_shape=jax.ShapeDtypeStruct((m, d), jnp.float32),
        in_specs=[
            pl.BlockSpec(memory_space=pltpu.VMEM),
            pl.BlockSpec(memory_space=pltpu.VMEM),
            pl.BlockSpec(memory_space=pltpu.VMEM),
        ],
        out_specs=pl.BlockSpec(memory_space=pltpu.VMEM),
        scratch_shapes=[
            pltpu.VMEM((qm, d), jnp.float32),
            pltpu.VMEM((qm, d), jnp.float32),
            pltpu.SemaphoreType.DMA((4,)),
            pltpu.SemaphoreType.DMA((4,)),
        ],
        compiler_params=pltpu.CompilerParams(collective_id=0),
    )(partial2d, resid, gamma2d)


# device time: 15517 ns/iter; 1.2660x vs baseline; 1.1210x over previous
import jax
import jax.numpy as jnp
from jax import lax
from jax.experimental import pallas as pl
from jax.experimental.pallas import tpu as pltpu

C = 4


def kernel(partial, resid, gamma):
    m, d = resid.shape
    qm = m // 4
    cm = qm // C
    gamma2d = gamma.reshape(1, d)

    def body(p_ref, r_ref, g_ref, out_ref, comm_ref, acc_ref,
             send_sems, recv_sems):
        my_x = lax.axis_index("x")
        my_y = lax.axis_index("y")
        my_z = lax.axis_index("z")
        q = 2 * (my_x ^ my_z) + (my_y ^ my_z)
        row0 = q * qm
        partner_z = (my_x, my_y, 1 - my_z)
        partner_x = (1 - my_x, my_y, my_z)
        partner_y = (my_x, 1 - my_y, my_z)

        barrier_sem = pltpu.get_barrier_semaphore()
        for nbr in (partner_z, partner_x, partner_y):
            pl.semaphore_signal(
                barrier_sem, inc=1, device_id=nbr,
                device_id_type=pl.DeviceIdType.MESH,
            )
        pl.semaphore_wait(barrier_sem, 3)

        ph1 = []
        for c in range(C):
            r = pltpu.make_async_remote_copy(
                src_ref=p_ref.at[0, pl.ds((3 - q) * qm + c * cm, cm), :],
                dst_ref=comm_ref.at[pl.ds(c * cm, cm), :],
                send_sem=send_sems.at[0, c],
                recv_sem=recv_sems.at[0, c],
                device_id=partner_z,
                device_id_type=pl.DeviceIdType.MESH,
            )
            r.start()
            ph1.append(r)

        acc_ref[...] = (
            p_ref[0, pl.ds(row0, qm), :] + r_ref[pl.ds(row0, qm), :]
        )

        ph2 = []
        for c in range(C):
            ph1[c].wait_recv()
            y = acc_ref[pl.ds(c * cm, cm), :] + comm_ref[pl.ds(c * cm, cm), :]
            rms = jnp.sqrt(jnp.mean(y * y, axis=-1, keepdims=True) + 1e-6)
            out_ref[pl.ds(row0 + c * cm, cm), :] = y / rms * g_ref[...]

            chunk = out_ref.at[pl.ds(row0 + c * cm, cm), :]
            for si, nbr in ((1, partner_z), (2, partner_x), (3, partner_y)):
                r = pltpu.make_async_remote_copy(
                    src_ref=chunk,
                    dst_ref=chunk,
                    send_sem=send_sems.at[si, c],
                    recv_sem=recv_sems.at[si, c],
                    device_id=nbr,
                    device_id_type=pl.DeviceIdType.MESH,
                )
                r.start()
                ph2.append(r)

        for si, qq in ((1, 3 - q), (2, q ^ 2), (3, q ^ 1)):
            for c in range(C):
                dst = out_ref.at[pl.ds(qq * qm + c * cm, cm), :]
                rr = pltpu.make_async_remote_copy(
                    src_ref=dst,
                    dst_ref=dst,
                    send_sem=send_sems.at[si, c],
                    recv_sem=recv_sems.at[si, c],
                    device_id=partner_z,
                    device_id_type=pl.DeviceIdType.MESH,
                )
                rr.wait_recv()
        for r in ph1:
            r.wait_send()
        for r in ph2:
            r.wait_send()

    return pl.pallas_call(
        body,
        out_shape=jax.ShapeDtypeStruct((m, d), jnp.float32),
        in_specs=[
            pl.BlockSpec(memory_space=pltpu.VMEM),
            pl.BlockSpec(memory_space=pltpu.VMEM),
            pl.BlockSpec(memory_space=pltpu.VMEM),
        ],
        out_specs=pl.BlockSpec(memory_space=pltpu.VMEM),
        scratch_shapes=[
            pltpu.VMEM((qm, d), jnp.float32),
            pltpu.VMEM((qm, d), jnp.float32),
            pltpu.SemaphoreType.DMA((4, C)),
            pltpu.SemaphoreType.DMA((4, C)),
        ],
        compiler_params=pltpu.CompilerParams(collective_id=0),
    )(partial, resid, gamma2d)


# device time: 14528 ns/iter; 1.3522x vs baseline; 1.0681x over previous
import jax
import jax.numpy as jnp
from jax import lax
from jax.experimental import pallas as pl
from jax.experimental.pallas import tpu as pltpu

C = 4


def kernel(partial, resid, gamma):
    m, d = resid.shape
    qm = m // 4
    cm = qm // C
    gamma2d = gamma.reshape(1, d)

    def body(p_ref, r_ref, g_ref, out_ref,
             comm_ref, acc_ref, psend_ref, pmine_ref, rq_ref, outq_ref,
             in_sems, out_sems, send_sems, recv_sems):
        my_x = lax.axis_index("x")
        my_y = lax.axis_index("y")
        my_z = lax.axis_index("z")
        q = 2 * (my_x ^ my_z) + (my_y ^ my_z)
        row0 = q * qm
        partner_z = (my_x, my_y, 1 - my_z)
        partner_x = (1 - my_x, my_y, my_z)
        partner_y = (my_x, 1 - my_y, my_z)

        cp_send = pltpu.make_async_copy(
            p_ref.at[0, pl.ds((3 - q) * qm, qm), :], psend_ref, in_sems.at[0])
        cp_mine = pltpu.make_async_copy(
            p_ref.at[0, pl.ds(row0, qm), :], pmine_ref, in_sems.at[1])
        cp_r = pltpu.make_async_copy(
            r_ref.at[pl.ds(row0, qm), :], rq_ref, in_sems.at[2])
        cp_send.start()
        cp_mine.start()
        cp_r.start()

        barrier_sem = pltpu.get_barrier_semaphore()
        for nbr in (partner_z, partner_x, partner_y):
            pl.semaphore_signal(
                barrier_sem, inc=1, device_id=nbr,
                device_id_type=pl.DeviceIdType.MESH,
            )
        pl.semaphore_wait(barrier_sem, 3)

        cp_send.wait()
        ph1 = []
        for c in range(C):
            r = pltpu.make_async_remote_copy(
                src_ref=psend_ref.at[pl.ds(c * cm, cm), :],
                dst_ref=comm_ref.at[pl.ds(c * cm, cm), :],
                send_sem=send_sems.at[0, c],
                recv_sem=recv_sems.at[0, c],
                device_id=partner_z,
                device_id_type=pl.DeviceIdType.MESH,
            )
            r.start()
            ph1.append(r)

        cp_mine.wait()
        cp_r.wait()
        acc_ref[...] = pmine_ref[...] + rq_ref[...]

        ph2 = []
        out_cps = []
        for c in range(C):
            ph1[c].wait_recv()
            y = acc_ref[pl.ds(c * cm, cm), :] + comm_ref[pl.ds(c * cm, cm), :]
            rms = jnp.sqrt(jnp.mean(y * y, axis=-1, keepdims=True) + 1e-6)
            outq_ref[pl.ds(c * cm, cm), :] = y / rms * g_ref[...]

            src = outq_ref.at[pl.ds(c * cm, cm), :]
            dst = out_ref.at[pl.ds(row0 + c * cm, cm), :]
            cp = pltpu.make_async_copy(src, dst, out_sems.at[c])
            cp.start()
            out_cps.append(cp)
            for si, nbr in ((1, partner_z), (2, partner_x), (3, partner_y)):
                r = pltpu.make_async_remote_copy(
                    src_ref=src,
                    dst_ref=dst,
                    send_sem=send_sems.at[si, c],
                    recv_sem=recv_sems.at[si, c],
                    device_id=nbr,
                    device_id_type=pl.DeviceIdType.MESH,
                )
                r.start()
                ph2.append(r)

        for si, qq in ((1, 3 - q), (2, q ^ 2), (3, q ^ 1)):
            for c in range(C):
                dst = out_ref.at[pl.ds(qq * qm + c * cm, cm), :]
                rr = pltpu.make_async_remote_copy(
                    src_ref=dst,
                    dst_ref=dst,
                    send_sem=send_sems.at[si, c],
                    recv_sem=recv_sems.at[si, c],
                    device_id=partner_z,
                    device_id_type=pl.DeviceIdType.MESH,
                )
                rr.wait_recv()
        for cp in out_cps:
            cp.wait()
        for r in ph1:
            r.wait_send()
        for r in ph2:
            r.wait_send()

    return pl.pallas_call(
        body,
        out_shape=jax.ShapeDtypeStruct((m, d), jnp.float32),
        in_specs=[
            pl.BlockSpec(memory_space=pl.ANY),
            pl.BlockSpec(memory_space=pl.ANY),
            pl.BlockSpec(memory_space=pltpu.VMEM),
        ],
        out_specs=pl.BlockSpec(memory_space=pl.ANY),
        scratch_shapes=[
            pltpu.VMEM((qm, d), jnp.float32),
            pltpu.VMEM((qm, d), jnp.float32),
            pltpu.VMEM((qm, d), jnp.float32),
            pltpu.VMEM((qm, d), jnp.float32),
            pltpu.VMEM((qm, d), jnp.float32),
            pltpu.VMEM((qm, d), jnp.float32),
            pltpu.SemaphoreType.DMA((3,)),
            pltpu.SemaphoreType.DMA((C,)),
            pltpu.SemaphoreType.DMA((4, C)),
            pltpu.SemaphoreType.DMA((4, C)),
        ],
        compiler_params=pltpu.CompilerParams(collective_id=0),
    )(partial, resid, gamma2d)
